# baseline (speedup 1.0000x reference)
import functools

import jax
import jax.numpy as jnp
from jax import lax
from jax.experimental import pallas as pl
from jax.experimental.pallas import tpu as pltpu
from jax.experimental.pallas import tpu_sc as plsc

_K = 1024
_D = 64
_T = 9216


def _dist_argmin_body(x_ref, e_ref, ind_ref, e2_ref):
    @pl.when(pl.program_id(0) == 0)
    def _():
        et = e_ref[...].T
        e2_ref[...] = jnp.sum(et * et, axis=0, keepdims=True)

    f = x_ref[...]
    ft = f.T
    f2 = jnp.sum(ft * ft, axis=0, keepdims=True).T
    mm = lax.dot_general(2.0 * f, e_ref[...], (((1,), (1,)), ((), ())),
                         preferred_element_type=jnp.float32)
    d = f2 - mm + e2_ref[...]
    ind_ref[...] = jnp.argmin(d, axis=1).astype(jnp.int32)


def _tc_argmin(flat, embed):
    n = flat.shape[0]
    return pl.pallas_call(
        _dist_argmin_body,
        grid=(n // _T,),
        in_specs=[
            pl.BlockSpec((_T, _D), lambda i: (i, 0)),
            pl.BlockSpec((_K, _D), lambda i: (0, 0)),
        ],
        out_specs=pl.BlockSpec((_T,), lambda i: (i,)),
        out_shape=jax.ShapeDtypeStruct((n,), jnp.int32),
        scratch_shapes=[pltpu.VMEM((1, _K), jnp.float32)],
    )(flat, embed)


_NC = 2
_NS = 16
_NW = _NC * _NS


@functools.cache
def _make_sc_gather(nrows, ch):
    bpw = nrows // _NW
    nch = bpw // ch

    @functools.partial(
        pl.kernel,
        mesh=plsc.VectorSubcoreMesh(core_axis_name="c", subcore_axis_name="s"),
        out_type=jax.ShapeDtypeStruct((nrows, _D), jnp.float32),
        compiler_params=pltpu.CompilerParams(use_tc_tiling_on_sc=False),
        scratch_types=[
            pltpu.VMEM((nch, ch), jnp.int32),
            pltpu.VMEM((bpw, _D), jnp.float32),
            pltpu.SemaphoreType.DMA,
            pltpu.SemaphoreType.DMA,
        ],
    )
    def _sc_gather(embed_hbm, idx_hbm, out_hbm, idx_v, rows_v, gsem, ssem):
        wid = lax.axis_index("s") * _NC + lax.axis_index("c")
        base = wid * bpw
        pltpu.sync_copy(idx_hbm.at[wid], idx_v)
        gathers = [pltpu.async_copy(embed_hbm.at[idx_v.at[j]],
                                    rows_v.at[pl.ds(j * ch, ch)], gsem)
                   for j in range(nch)]
        stores = []
        for j in range(nch):
            gathers[j].wait()
            stores.append(pltpu.async_copy(
                rows_v.at[pl.ds(j * ch, ch)],
                out_hbm.at[pl.ds(base + j * ch, ch)], ssem))
        for s in stores:
            s.wait()

    return _sc_gather


def kernel(x, embed):
    shape = x.shape
    flat = x.reshape(-1, shape[-1])
    ind = _tc_argmin(flat, embed)
    quant = _make_sc_gather(flat.shape[0], 96)(embed, ind.reshape(_NW, 3, 96))
    return quant.reshape(shape), ind.reshape(shape[:-1])


# speedup vs baseline: 1.0283x; 1.0283x over previous
import functools

import jax
import jax.numpy as jnp
from jax import lax
from jax.experimental import pallas as pl
from jax.experimental.pallas import tpu as pltpu
from jax.experimental.pallas import tpu_sc as plsc

_K = 1024
_D = 64
_T = 9216


def _dist_argmin_body(x_ref, e_ref, ind_ref, ind2d_ref, e2_ref):
    @pl.when(pl.program_id(0) == 0)
    def _():
        et = e_ref[...].T
        e2_ref[...] = jnp.sum(et * et, axis=0, keepdims=True)

    f = x_ref[...]
    ft = f.T
    f2 = jnp.sum(ft * ft, axis=0, keepdims=True).T
    mm = lax.dot_general(2.0 * f, e_ref[...], (((1,), (1,)), ((), ())),
                         preferred_element_type=jnp.float32)
    d = f2 - mm + e2_ref[...]
    ind = jnp.argmin(d, axis=1).astype(jnp.int32)
    ind_ref[...] = ind
    ind2d_ref[...] = ind.reshape(ind2d_ref.shape)


def _tc_argmin(flat, embed, lead_shape):
    n = flat.shape[0]
    return pl.pallas_call(
        _dist_argmin_body,
        grid=(n // _T,),
        in_specs=[
            pl.BlockSpec((_T, _D), lambda i: (i, 0)),
            pl.BlockSpec((_K, _D), lambda i: (0, 0)),
        ],
        out_specs=[
            pl.BlockSpec((_T,), lambda i: (i,)),
            pl.BlockSpec(lead_shape, lambda i: (0, 0)),
        ],
        out_shape=[
            jax.ShapeDtypeStruct((n,), jnp.int32),
            jax.ShapeDtypeStruct(lead_shape, jnp.int32),
        ],
        scratch_shapes=[pltpu.VMEM((1, _K), jnp.float32)],
    )(flat, embed)


_NC = 2
_NS = 16
_NW = _NC * _NS


@functools.cache
def _make_sc_gather(nrows, ch):
    bpw = nrows // _NW
    nch = bpw // ch

    @functools.partial(
        pl.kernel,
        mesh=plsc.VectorSubcoreMesh(core_axis_name="c", subcore_axis_name="s"),
        out_type=jax.ShapeDtypeStruct((nrows, _D), jnp.float32),
        compiler_params=pltpu.CompilerParams(use_tc_tiling_on_sc=False),
        scratch_types=[
            pltpu.VMEM((nch, ch), jnp.int32),
            pltpu.VMEM((bpw, _D), jnp.float32),
            pltpu.SemaphoreType.DMA,
            pltpu.SemaphoreType.DMA,
        ],
    )
    def _sc_gather(embed_hbm, idx_hbm, out_hbm, idx_v, rows_v, gsem, ssem):
        wid = lax.axis_index("s") * _NC + lax.axis_index("c")
        base = wid * bpw
        pltpu.sync_copy(idx_hbm.at[wid], idx_v)
        gathers = [pltpu.async_copy(embed_hbm.at[idx_v.at[j]],
                                    rows_v.at[pl.ds(j * ch, ch)], gsem)
                   for j in range(nch)]
        stores = []
        for j in range(nch):
            gathers[j].wait()
            stores.append(pltpu.async_copy(
                rows_v.at[pl.ds(j * ch, ch)],
                out_hbm.at[pl.ds(base + j * ch, ch)], ssem))
        for s in stores:
            s.wait()

    return _sc_gather


def kernel(x, embed):
    shape = x.shape
    flat = x.reshape(-1, shape[-1])
    ind, ind2d = _tc_argmin(flat, embed, shape[:-1])
    quant = _make_sc_gather(flat.shape[0], 96)(embed, ind.reshape(_NW, 3, 96))
    return quant.reshape(shape), ind2d
